# baseline (speedup 1.0000x reference)
import functools

import jax
import jax.numpy as jnp
from jax import lax
from jax.experimental import pallas as pl
from jax.experimental.pallas import tpu as pltpu
from jax.experimental.pallas import tpu_sc as plsc

C_IN = 96
C_OUT = 96
KH = KW = 3
NTAP = KH * KW
CK = KH * C_IN
WFLAT = KW * C_OUT * CK


def _sc_build_w(nnz, rows_hbm, cols_hbm, vals_hbm, zeros_hbm, w_hbm,
                rows_v, cols_v, vals_v, wbuf):
    core = lax.axis_index("c")
    sub = lax.axis_index("s")

    @pl.when(core * 16 + sub == 0)
    def _():
        pltpu.sync_copy(rows_hbm, rows_v.at[pl.ds(0, nnz)])
        pltpu.sync_copy(cols_hbm, cols_v.at[pl.ds(0, nnz)])
        pltpu.sync_copy(vals_hbm, vals_v.at[pl.ds(0, nnz)])
        pltpu.sync_copy(zeros_hbm, wbuf)

        nfull = nnz // 16
        tail = nnz % 16
        UN = 4

        def flat_idx(r, c):
            cin = (c * 7282) >> 16
            t = c - cin * NTAP
            kh = (t * 11) >> 5
            kw = t - kh * KW
            return (kw * C_OUT + r) * CK + kh * C_IN + cin

        def scat16(b):
            r = rows_v[pl.ds(b, 16)]
            c = cols_v[pl.ds(b, 16)]
            v = vals_v[pl.ds(b, 16)]
            plsc.store_scatter(wbuf, [flat_idx(r, c)], v)

        def scat_body(i, carry):
            for k in range(UN):
                scat16(i * (16 * UN) + k * 16)
            return carry

        lax.fori_loop(0, nfull // UN, scat_body, 0)
        for j in range((nfull // UN) * UN, nfull):
            scat16(j * 16)

        if tail:
            b = nfull * 16
            r = rows_v[pl.ds(b, 16)]
            c = cols_v[pl.ds(b, 16)]
            v = vals_v[pl.ds(b, 16)]
            m = lax.iota(jnp.int32, 16) < tail
            plsc.store_scatter(wbuf, [flat_idx(r, c)], v, mask=m)

        pltpu.sync_copy(wbuf, w_hbm)


def _build_w_stack(w_rows, w_cols, w_vals):
    if True:
        Wd = jnp.zeros((C_OUT, 864), jnp.float32).at[w_rows, w_cols].set(w_vals)
        return jnp.transpose(Wd.reshape(C_OUT, C_IN, KH, KW),
                             (3, 0, 2, 1)).reshape(KW, C_OUT, CK)
    nnz = w_rows.shape[0]
    nnz_pad = ((nnz + 15) // 16) * 16
    mesh = plsc.VectorSubcoreMesh(core_axis_name="c", subcore_axis_name="s")
    w_flat = pl.kernel(
        functools.partial(_sc_build_w, nnz),
        out_type=jax.ShapeDtypeStruct((WFLAT,), jnp.float32),
        mesh=mesh,
        compiler_params=pltpu.CompilerParams(needs_layout_passes=False),
        scratch_types=[
            pltpu.VMEM((nnz_pad,), jnp.int32),
            pltpu.VMEM((nnz_pad,), jnp.int32),
            pltpu.VMEM((nnz_pad,), jnp.float32),
            pltpu.VMEM((WFLAT,), jnp.float32),
        ],
    )(w_rows, w_cols, w_vals, jnp.zeros((WFLAT,), jnp.float32))
    return w_flat.reshape(KW, C_OUT, CK)


def _conv_body(x0, x1, x2, x3, x4, x5, x6, x7, x8,
               sel_ref, w_ref, b_ref, out_ref):
    w_out = out_ref.shape[4]
    nh = out_ref.shape[2]
    padw = sel_ref.shape[1] // 2
    h = pl.program_id(1)
    row_valid = jnp.where(h > 0, 1.0, 0.0).astype(jnp.float32)
    xs = (x0, x1, x2, x3, x4, x5, x6, x7, x8)
    r = [ref[0, :, 0, 0, :] for ref in xs]
    r[0] = r[0] * row_valid
    rows = jnp.concatenate(
        sum([[r[2 * q], r[2 * q + 1], r[2 * q + 2]] for q in range(nh)], []),
        axis=0)
    dims = (((1,), (0,)), ((), ()))
    par = lax.dot_general(rows, sel_ref[...], dims,
                          preferred_element_type=jnp.float32)
    for q in range(nh):
        lo = q * CK
        p0 = par[lo:lo + CK, 0:w_out]
        p1 = par[lo:lo + CK, padw:padw + w_out]
        m0 = lax.dot_general(w_ref[0], p1, dims,
                             preferred_element_type=jnp.float32)
        m1 = lax.dot_general(w_ref[1], p0, dims,
                             preferred_element_type=jnp.float32)
        m2 = lax.dot_general(w_ref[2], p1, dims,
                             preferred_element_type=jnp.float32)
        shifted = jnp.pad(m0, ((0, 0), (1, 0)))[:, :w_out]
        out_ref[0, :, q, 0, :] = m1 + m2 + shifted + b_ref[...]


def kernel(x, w_rows, w_cols, w_vals, bias):
    B, C, H, W = x.shape
    H_out, W_out = H // 2, W // 2
    PADW = 128 * ((W_out + 127) // 128)

    w_stack = _build_w_stack(w_rows, w_cols, w_vals)
    bias2 = bias.reshape(C_OUT, 1)

    j = jnp.arange(W)[:, None]
    wcol = jnp.arange(W_out)[None, :]
    pad_tail = ((0, 0), (0, PADW - W_out))
    sel_e = jnp.pad((j == 2 * wcol).astype(jnp.float32), pad_tail)
    sel_o = jnp.pad((j == 2 * wcol + 1).astype(jnp.float32), pad_tail)
    sel2 = jnp.concatenate([sel_e, sel_o], axis=1)

    x5 = x.reshape(B, C, H, 1, W)

    def xspec(kh):
        return pl.BlockSpec(
            (1, C, 1, 1, W),
            lambda b, h, kh=kh: (b, 0, jnp.maximum(8 * h + kh - 1, 0), 0, 0))

    out5 = pl.pallas_call(
        _conv_body,
        grid=(B, H_out // 4),
        in_specs=[
            xspec(0), xspec(1), xspec(2), xspec(3), xspec(4),
            xspec(5), xspec(6), xspec(7), xspec(8),
            pl.BlockSpec((W, 2 * PADW), lambda b, h: (0, 0)),
            pl.BlockSpec((KW, C_OUT, CK), lambda b, h: (0, 0, 0)),
            pl.BlockSpec((C_OUT, 1), lambda b, h: (0, 0)),
        ],
        out_specs=pl.BlockSpec((1, C_OUT, 4, 1, W_out),
                               lambda b, h: (b, 0, h, 0, 0)),
        out_shape=jax.ShapeDtypeStruct((B, C_OUT, H_out, 1, W_out),
                                       jnp.float32),
        compiler_params=pltpu.CompilerParams(
            dimension_semantics=("arbitrary", "arbitrary")),
    )(x5, x5, x5, x5, x5, x5, x5, x5, x5, sel2, w_stack, bias2)

    return out5.reshape(B, C_OUT, H_out, W_out)


# speedup vs baseline: 1.2267x; 1.2267x over previous
import functools

import jax
import jax.numpy as jnp
from jax import lax
from jax.experimental import pallas as pl
from jax.experimental.pallas import tpu as pltpu
from jax.experimental.pallas import tpu_sc as plsc

C_IN = 96
C_OUT = 96
KH = KW = 3
NTAP = KH * KW
CK = KH * C_IN
WFLAT = KW * C_OUT * CK


def _sc_build_w(nnz, rows_hbm, cols_hbm, vals_hbm, zeros_hbm, w_hbm,
                rows_v, cols_v, vals_v, wbuf):
    core = lax.axis_index("c")
    sub = lax.axis_index("s")

    @pl.when(core * 16 + sub == 0)
    def _():
        pltpu.sync_copy(rows_hbm, rows_v.at[pl.ds(0, nnz)])
        pltpu.sync_copy(cols_hbm, cols_v.at[pl.ds(0, nnz)])
        pltpu.sync_copy(vals_hbm, vals_v.at[pl.ds(0, nnz)])
        pltpu.sync_copy(zeros_hbm, wbuf)

        nfull = nnz // 16
        tail = nnz % 16
        UN = 4

        def flat_idx(r, c):
            cin = (c * 7282) >> 16
            t = c - cin * NTAP
            kh = (t * 11) >> 5
            kw = t - kh * KW
            return (kw * C_OUT + r) * CK + kh * C_IN + cin

        def scat16(b):
            r = rows_v[pl.ds(b, 16)]
            c = cols_v[pl.ds(b, 16)]
            v = vals_v[pl.ds(b, 16)]
            plsc.store_scatter(wbuf, [flat_idx(r, c)], v)

        def scat_body(i, carry):
            for k in range(UN):
                scat16(i * (16 * UN) + k * 16)
            return carry

        lax.fori_loop(0, nfull // UN, scat_body, 0)
        for j in range((nfull // UN) * UN, nfull):
            scat16(j * 16)

        if tail:
            b = nfull * 16
            r = rows_v[pl.ds(b, 16)]
            c = cols_v[pl.ds(b, 16)]
            v = vals_v[pl.ds(b, 16)]
            m = lax.iota(jnp.int32, 16) < tail
            plsc.store_scatter(wbuf, [flat_idx(r, c)], v, mask=m)

        pltpu.sync_copy(wbuf, w_hbm)


def _build_w_stack(w_rows, w_cols, w_vals):
    nnz = w_rows.shape[0]
    nnz_pad = ((nnz + 15) // 16) * 16
    mesh = plsc.VectorSubcoreMesh(core_axis_name="c", subcore_axis_name="s")
    w_flat = pl.kernel(
        functools.partial(_sc_build_w, nnz),
        out_type=jax.ShapeDtypeStruct((WFLAT,), jnp.float32),
        mesh=mesh,
        compiler_params=pltpu.CompilerParams(needs_layout_passes=False),
        scratch_types=[
            pltpu.VMEM((nnz_pad,), jnp.int32),
            pltpu.VMEM((nnz_pad,), jnp.int32),
            pltpu.VMEM((nnz_pad,), jnp.float32),
            pltpu.VMEM((WFLAT,), jnp.float32),
        ],
    )(w_rows, w_cols, w_vals, jnp.zeros((WFLAT,), jnp.float32))
    return w_flat.reshape(KW, C_OUT, CK)


def _conv_body(x0, x1, x2, x3, x4, x5, x6, x7, x8,
               sel_ref, w_ref, b_ref, out_ref):
    w_out = out_ref.shape[4]
    nh = out_ref.shape[2]
    padw = sel_ref.shape[1] // 2
    h = pl.program_id(1)
    row_valid = jnp.where(h > 0, 1.0, 0.0).astype(jnp.float32)
    xs = (x0, x1, x2, x3, x4, x5, x6, x7, x8)
    r = [ref[0, :, 0, 0, :] for ref in xs]
    r[0] = r[0] * row_valid
    rows = jnp.concatenate(
        sum([[r[2 * q], r[2 * q + 1], r[2 * q + 2]] for q in range(nh)], []),
        axis=0)
    dims = (((1,), (0,)), ((), ()))
    par = lax.dot_general(rows.astype(jnp.bfloat16), sel_ref[...], dims,
                          preferred_element_type=jnp.float32)
    for q in range(nh):
        lo = q * CK
        p0 = par[lo:lo + CK, 0:w_out].astype(jnp.bfloat16)
        p1 = par[lo:lo + CK, padw:padw + w_out].astype(jnp.bfloat16)
        m0 = lax.dot_general(w_ref[0], p1, dims,
                             preferred_element_type=jnp.float32)
        m1 = lax.dot_general(w_ref[1], p0, dims,
                             preferred_element_type=jnp.float32)
        m2 = lax.dot_general(w_ref[2], p1, dims,
                             preferred_element_type=jnp.float32)
        shifted = jnp.pad(m0, ((0, 0), (1, 0)))[:, :w_out]
        out_ref[0, :, q, 0, :] = m1 + m2 + shifted + b_ref[...]


def kernel(x, w_rows, w_cols, w_vals, bias):
    B, C, H, W = x.shape
    H_out, W_out = H // 2, W // 2
    PADW = 128 * ((W_out + 127) // 128)

    w_stack = _build_w_stack(w_rows, w_cols, w_vals).astype(jnp.bfloat16)
    bias2 = bias.reshape(C_OUT, 1)

    j = jnp.arange(W)[:, None]
    wcol = jnp.arange(W_out)[None, :]
    pad_tail = ((0, 0), (0, PADW - W_out))
    sel_e = jnp.pad((j == 2 * wcol).astype(jnp.bfloat16), pad_tail)
    sel_o = jnp.pad((j == 2 * wcol + 1).astype(jnp.bfloat16), pad_tail)
    sel2 = jnp.concatenate([sel_e, sel_o], axis=1)

    x5 = x.reshape(B, C, H, 1, W)

    def xspec(kh):
        return pl.BlockSpec(
            (1, C, 1, 1, W),
            lambda b, h, kh=kh: (b, 0, jnp.maximum(8 * h + kh - 1, 0), 0, 0))

    out5 = pl.pallas_call(
        _conv_body,
        grid=(B, H_out // 4),
        in_specs=[
            xspec(0), xspec(1), xspec(2), xspec(3), xspec(4),
            xspec(5), xspec(6), xspec(7), xspec(8),
            pl.BlockSpec((W, 2 * PADW), lambda b, h: (0, 0)),
            pl.BlockSpec((KW, C_OUT, CK), lambda b, h: (0, 0, 0)),
            pl.BlockSpec((C_OUT, 1), lambda b, h: (0, 0)),
        ],
        out_specs=pl.BlockSpec((1, C_OUT, 4, 1, W_out),
                               lambda b, h: (b, 0, h, 0, 0)),
        out_shape=jax.ShapeDtypeStruct((B, C_OUT, H_out, 1, W_out),
                                       jnp.float32),
        compiler_params=pltpu.CompilerParams(
            dimension_semantics=("arbitrary", "arbitrary")),
    )(x5, x5, x5, x5, x5, x5, x5, x5, x5, sel2, w_stack, bias2)

    return out5.reshape(B, C_OUT, H_out, W_out)


# speedup vs baseline: 1.4405x; 1.1742x over previous
import functools

import jax
import jax.numpy as jnp
from jax import lax
from jax.experimental import pallas as pl
from jax.experimental.pallas import tpu as pltpu
from jax.experimental.pallas import tpu_sc as plsc

C_IN = 96
C_OUT = 96
KH = KW = 3
NTAP = KH * KW
CK = KH * C_IN
WFLAT = KW * C_OUT * CK


def _sc_build_w(nnz, rows_hbm, cols_hbm, vals_hbm, zeros_hbm, w_hbm,
                rows_v, cols_v, vals_v, wbuf):
    core = lax.axis_index("c")
    sub = lax.axis_index("s")

    @pl.when(core * 16 + sub == 0)
    def _():
        pltpu.sync_copy(rows_hbm, rows_v.at[pl.ds(0, nnz)])
        pltpu.sync_copy(cols_hbm, cols_v.at[pl.ds(0, nnz)])
        pltpu.sync_copy(vals_hbm, vals_v.at[pl.ds(0, nnz)])
        pltpu.sync_copy(zeros_hbm, wbuf)

        nfull = nnz // 16
        tail = nnz % 16
        UN = 4

        def flat_idx(r, c):
            cin = (c * 7282) >> 16
            t = c - cin * NTAP
            kh = (t * 11) >> 5
            kw = t - kh * KW
            return (kw * C_OUT + r) * CK + kh * C_IN + cin

        def scat16(b):
            r = rows_v[pl.ds(b, 16)]
            c = cols_v[pl.ds(b, 16)]
            v = vals_v[pl.ds(b, 16)]
            plsc.store_scatter(wbuf, [flat_idx(r, c)], v)

        def scat_body(i, carry):
            for k in range(UN):
                scat16(i * (16 * UN) + k * 16)
            return carry

        lax.fori_loop(0, nfull // UN, scat_body, 0)
        for j in range((nfull // UN) * UN, nfull):
            scat16(j * 16)

        if tail:
            b = nfull * 16
            r = rows_v[pl.ds(b, 16)]
            c = cols_v[pl.ds(b, 16)]
            v = vals_v[pl.ds(b, 16)]
            m = lax.iota(jnp.int32, 16) < tail
            plsc.store_scatter(wbuf, [flat_idx(r, c)], v, mask=m)

        pltpu.sync_copy(wbuf, w_hbm)


def _build_w_stack(w_rows, w_cols, w_vals):
    nnz = w_rows.shape[0]
    nnz_pad = ((nnz + 15) // 16) * 16
    mesh = plsc.VectorSubcoreMesh(core_axis_name="c", subcore_axis_name="s")
    w_flat = pl.kernel(
        functools.partial(_sc_build_w, nnz),
        out_type=jax.ShapeDtypeStruct((WFLAT,), jnp.float32),
        mesh=mesh,
        compiler_params=pltpu.CompilerParams(needs_layout_passes=False),
        scratch_types=[
            pltpu.VMEM((nnz_pad,), jnp.int32),
            pltpu.VMEM((nnz_pad,), jnp.int32),
            pltpu.VMEM((nnz_pad,), jnp.float32),
            pltpu.VMEM((WFLAT,), jnp.float32),
        ],
    )(w_rows, w_cols, w_vals, jnp.zeros((WFLAT,), jnp.float32))
    return w_flat.reshape(KW, C_OUT, CK)


def _conv_body(*args):
    sel_ref, w_ref, b_ref, out_ref = args[-4:]
    xs = args[:-4]
    w_out = out_ref.shape[4]
    nh = out_ref.shape[2]
    padw = sel_ref.shape[1] // 2
    h = pl.program_id(1)
    row_valid = jnp.where(h > 0, 1.0, 0.0).astype(jnp.float32)
    r = [ref[0, :, 0, 0, :] for ref in xs]
    r[0] = r[0] * row_valid
    rows = jnp.concatenate(
        sum([[r[2 * q], r[2 * q + 1], r[2 * q + 2]] for q in range(nh)], []),
        axis=0)
    dims = (((1,), (0,)), ((), ()))
    par = lax.dot_general(rows, sel_ref[...], dims,
                          preferred_element_type=jnp.float32)
    for q in range(nh):
        lo = q * CK
        p0 = par[lo:lo + CK, 0:w_out]
        p1 = par[lo:lo + CK, padw:padw + w_out]
        m0 = lax.dot_general(w_ref[0], p1, dims,
                             preferred_element_type=jnp.float32)
        m1 = lax.dot_general(w_ref[1], p0, dims,
                             preferred_element_type=jnp.float32)
        m2 = lax.dot_general(w_ref[2], p1, dims,
                             preferred_element_type=jnp.float32)
        shifted = jnp.pad(m0, ((0, 0), (1, 0)))[:, :w_out]
        out_ref[0, :, q, 0, :] = m1 + m2 + shifted + b_ref[...]


def kernel(x, w_rows, w_cols, w_vals, bias):
    B, C, H, W = x.shape
    H_out, W_out = H // 2, W // 2
    PADW = 128 * ((W_out + 127) // 128)

    w_stack = _build_w_stack(w_rows, w_cols, w_vals)
    bias2 = bias.reshape(C_OUT, 1)

    j = jnp.arange(W)[:, None]
    wcol = jnp.arange(W_out)[None, :]
    pad_tail = ((0, 0), (0, PADW - W_out))
    sel_e = jnp.pad((j == 2 * wcol).astype(jnp.float32), pad_tail)
    sel_o = jnp.pad((j == 2 * wcol + 1).astype(jnp.float32), pad_tail)
    sel2 = jnp.concatenate([sel_e, sel_o], axis=1)

    xr = x.reshape(B, C, H, 1, W)
    NH = 8
    nrefs = 2 * NH + 1

    def xspec(kh):
        return pl.BlockSpec(
            (1, C, 1, 1, W),
            lambda b, h, kh=kh: (
                b, 0, jnp.maximum(2 * NH * h + kh - 1, 0), 0, 0))

    out5 = pl.pallas_call(
        _conv_body,
        grid=(B, H_out // NH),
        in_specs=[xspec(k) for k in range(nrefs)] + [
            pl.BlockSpec((W, 2 * PADW), lambda b, h: (0, 0)),
            pl.BlockSpec((KW, C_OUT, CK), lambda b, h: (0, 0, 0)),
            pl.BlockSpec((C_OUT, 1), lambda b, h: (0, 0)),
        ],
        out_specs=pl.BlockSpec((1, C_OUT, NH, 1, W_out),
                               lambda b, h: (b, 0, h, 0, 0)),
        out_shape=jax.ShapeDtypeStruct((B, C_OUT, H_out, 1, W_out),
                                       jnp.float32),
        compiler_params=pltpu.CompilerParams(
            dimension_semantics=("arbitrary", "arbitrary")),
    )(*([xr] * nrefs), sel2, w_stack, bias2)

    return out5.reshape(B, C_OUT, H_out, W_out)


# speedup vs baseline: 1.4837x; 1.0300x over previous
import functools

import jax
import jax.numpy as jnp
from jax import lax
from jax.experimental import pallas as pl
from jax.experimental.pallas import tpu as pltpu
from jax.experimental.pallas import tpu_sc as plsc

C_IN = 96
C_OUT = 96
KH = KW = 3
NTAP = KH * KW
CK = KH * C_IN
WFLAT = KW * C_OUT * CK


def _sc_build_w(nnz, rows_hbm, cols_hbm, vals_hbm, zeros_hbm, w_hbm,
                rows_v, cols_v, vals_v, wbuf):
    core = lax.axis_index("c")
    sub = lax.axis_index("s")

    @pl.when(core * 16 + sub == 0)
    def _():
        pltpu.sync_copy(rows_hbm, rows_v.at[pl.ds(0, nnz)])
        pltpu.sync_copy(cols_hbm, cols_v.at[pl.ds(0, nnz)])
        pltpu.sync_copy(vals_hbm, vals_v.at[pl.ds(0, nnz)])
        pltpu.sync_copy(zeros_hbm, wbuf)

        nfull = nnz // 16
        tail = nnz % 16
        UN = 4

        def flat_idx(r, c):
            cin = (c * 7282) >> 16
            t = c - cin * NTAP
            kh = (t * 11) >> 5
            kw = t - kh * KW
            return (kw * C_OUT + r) * CK + kh * C_IN + cin

        def scat16(b):
            r = rows_v[pl.ds(b, 16)]
            c = cols_v[pl.ds(b, 16)]
            v = vals_v[pl.ds(b, 16)]
            plsc.store_scatter(wbuf, [flat_idx(r, c)], v)

        def scat_body(i, carry):
            for k in range(UN):
                scat16(i * (16 * UN) + k * 16)
            return carry

        lax.fori_loop(0, nfull // UN, scat_body, 0)
        for j in range((nfull // UN) * UN, nfull):
            scat16(j * 16)

        if tail:
            b = nfull * 16
            r = rows_v[pl.ds(b, 16)]
            c = cols_v[pl.ds(b, 16)]
            v = vals_v[pl.ds(b, 16)]
            m = lax.iota(jnp.int32, 16) < tail
            plsc.store_scatter(wbuf, [flat_idx(r, c)], v, mask=m)

        pltpu.sync_copy(wbuf, w_hbm)


def _build_w_stack(w_rows, w_cols, w_vals):
    nnz = w_rows.shape[0]
    nnz_pad = ((nnz + 15) // 16) * 16
    mesh = plsc.VectorSubcoreMesh(core_axis_name="c", subcore_axis_name="s")
    w_flat = pl.kernel(
        functools.partial(_sc_build_w, nnz),
        out_type=jax.ShapeDtypeStruct((WFLAT,), jnp.float32),
        mesh=mesh,
        compiler_params=pltpu.CompilerParams(needs_layout_passes=False),
        scratch_types=[
            pltpu.VMEM((nnz_pad,), jnp.int32),
            pltpu.VMEM((nnz_pad,), jnp.int32),
            pltpu.VMEM((nnz_pad,), jnp.float32),
            pltpu.VMEM((WFLAT,), jnp.float32),
        ],
    )(w_rows, w_cols, w_vals, jnp.zeros((WFLAT,), jnp.float32))
    return w_flat.reshape(KW, C_OUT, CK)


def _conv_body(*args):
    sel_ref, w_ref, b_ref, out_ref = args[-4:]
    xs = args[:-4]
    w_out = out_ref.shape[4]
    nh = out_ref.shape[2]
    padw = sel_ref.shape[1] // 2
    h = pl.program_id(1)
    row_valid = jnp.where(h > 0, 1.0, 0.0).astype(jnp.float32)
    r = [ref[0, :, 0, 0, :] for ref in xs]
    r[0] = r[0] * row_valid
    rows = jnp.concatenate(
        sum([[r[2 * q], r[2 * q + 1], r[2 * q + 2]] for q in range(nh)], []),
        axis=0)
    dims = (((1,), (0,)), ((), ()))
    par = lax.dot_general(rows, sel_ref[...], dims,
                          preferred_element_type=jnp.float32)
    for q in range(nh):
        lo = q * CK
        p0 = par[lo:lo + CK, 0:w_out]
        p1 = par[lo:lo + CK, padw:padw + w_out]
        m0 = lax.dot_general(w_ref[0], p1, dims,
                             preferred_element_type=jnp.float32)
        m1 = lax.dot_general(w_ref[1], p0, dims,
                             preferred_element_type=jnp.float32)
        m2 = lax.dot_general(w_ref[2], p1, dims,
                             preferred_element_type=jnp.float32)
        shifted = jnp.pad(m0, ((0, 0), (1, 0)))[:, :w_out]
        out_ref[0, :, q, 0, :] = m1 + m2 + shifted + b_ref[...]


def kernel(x, w_rows, w_cols, w_vals, bias):
    B, C, H, W = x.shape
    H_out, W_out = H // 2, W // 2
    PADW = 128 * ((W_out + 127) // 128)

    w_stack = _build_w_stack(w_rows, w_cols, w_vals)
    bias2 = bias.reshape(C_OUT, 1)

    j = jnp.arange(W)[:, None]
    wcol = jnp.arange(W_out)[None, :]
    pad_tail = ((0, 0), (0, PADW - W_out))
    sel_e = jnp.pad((j == 2 * wcol).astype(jnp.float32), pad_tail)
    sel_o = jnp.pad((j == 2 * wcol + 1).astype(jnp.float32), pad_tail)
    sel2 = jnp.concatenate([sel_e, sel_o], axis=1)

    xr = x.reshape(B, C, H, 1, W)
    NH = 16
    nrefs = 2 * NH + 1

    def xspec(kh):
        return pl.BlockSpec(
            (1, C, 1, 1, W),
            lambda b, h, kh=kh: (
                b, 0, jnp.maximum(2 * NH * h + kh - 1, 0), 0, 0))

    out5 = pl.pallas_call(
        _conv_body,
        grid=(B, H_out // NH),
        in_specs=[xspec(k) for k in range(nrefs)] + [
            pl.BlockSpec((W, 2 * PADW), lambda b, h: (0, 0)),
            pl.BlockSpec((KW, C_OUT, CK), lambda b, h: (0, 0, 0)),
            pl.BlockSpec((C_OUT, 1), lambda b, h: (0, 0)),
        ],
        out_specs=pl.BlockSpec((1, C_OUT, NH, 1, W_out),
                               lambda b, h: (b, 0, h, 0, 0)),
        out_shape=jax.ShapeDtypeStruct((B, C_OUT, H_out, 1, W_out),
                                       jnp.float32),
        compiler_params=pltpu.CompilerParams(
            dimension_semantics=("arbitrary", "arbitrary")),
    )(*([xr] * nrefs), sel2, w_stack, bias2)

    return out5.reshape(B, C_OUT, H_out, W_out)
